# baseline (speedup 1.0000x reference)
import functools

import jax
import jax.numpy as jnp
from jax import lax
from jax.experimental import pallas as pl
from jax.experimental.pallas import tpu as pltpu
from jax.experimental.pallas import tpu_sc as plsc

B = 16384
D = 32
ROWS_PER_LINE = 4
NLINES = 250000
NC = 2
NS = 16
L = 16
NW = NC * NS
BPW = B // NW
IDXW = 128
NIDX = BPW // IDXW
WAVE = 2
NWAVES = NIDX // WAVE
GLOBAL_MEAN = 3.82

_mesh = plsc.VectorSubcoreMesh(core_axis_name="c", subcore_axis_name="s")


@functools.partial(
    pl.kernel,
    mesh=_mesh,
    compiler_params=pltpu.CompilerParams(
        needs_layout_passes=False, use_tc_tiling_on_sc=True),
    out_type=jax.ShapeDtypeStruct((B,), jnp.float32),
    scratch_types=[
        pltpu.VMEM((NIDX, IDXW), jnp.int32),
        pltpu.VMEM((NIDX, IDXW), jnp.int32),
        pltpu.VMEM((NIDX, IDXW), jnp.int32),
        pltpu.VMEM((NIDX, IDXW), jnp.int32),
        pltpu.VMEM((WAVE * IDXW, 128), jnp.float32),
        pltpu.VMEM((WAVE * IDXW, 128), jnp.float32),
        pltpu.VMEM((BPW,), jnp.float32),
        pltpu.VMEM((BPW,), jnp.float32),
        pltpu.VMEM((BPW,), jnp.float32),
        pltpu.SemaphoreType.DMA,
    ],
)
def _svdpp_sc(sid_hbm, pid_hbm, sf_hbm, pf_hbm, sb_hbm, pb_hbm, out_hbm,
              sid_v, pid_v, sq_v, pq_v, srows_v, prows_v, sb_v, pb_v,
              out_v, sem):
    wid = lax.axis_index("s") * NC + lax.axis_index("c")
    base = wid * BPW

    for j in range(NIDX):
        bb = pl.ds(0, IDXW)
        blk = wid * NIDX + j
        pltpu.sync_copy(sid_hbm.at[blk // 8, blk % 8], sid_v.at[j])
        pltpu.sync_copy(pid_hbm.at[blk // 8, blk % 8], pid_v.at[j])

    def lidx_body(t, carry):
        j = t // 8
        o = (t - j * 8) * L
        sv = sid_v[j, pl.ds(o, L)]
        pv = pid_v[j, pl.ds(o, L)]
        sq_v[j, pl.ds(o, L)] = sv // ROWS_PER_LINE
        pq_v[j, pl.ds(o, L)] = pv // ROWS_PER_LINE
        return carry

    lax.fori_loop(0, NIDX * 8, lidx_body, 0)

    bias_copies = []
    for j in range(NIDX):
        dst = pl.ds(j * IDXW, IDXW)
        bias_copies.append(
            pltpu.async_copy(sb_hbm.at[sid_v.at[j]], sb_v.at[dst], sem))
        bias_copies.append(
            pltpu.async_copy(pb_hbm.at[pid_v.at[j]], pb_v.at[dst], sem))

    lanes = lax.iota(jnp.int32, L)

    for w in range(NWAVES):
        copies = []
        for jj in range(WAVE):
            j = w * WAVE + jj
            dst = pl.ds(jj * IDXW, IDXW)
            copies.append(pltpu.async_copy(
                sf_hbm.at[sq_v.at[j]], srows_v.at[dst], sem))
            copies.append(pltpu.async_copy(
                pf_hbm.at[pq_v.at[j]], prows_v.at[dst], sem))
        for c in copies:
            c.wait()

        def group_body(g, carry):
            rloc = g * L + lanes
            j = w * WAVE + g // (IDXW // L)
            o = (g - (g // (IDXW // L)) * (IDXW // L)) * L
            sv_ids = sid_v[j, pl.ds(o, L)]
            pv_ids = pid_v[j, pl.ds(o, L)]
            scol0 = (sv_ids - (sv_ids // ROWS_PER_LINE) * ROWS_PER_LINE) * D
            pcol0 = (pv_ids - (pv_ids // ROWS_PER_LINE) * ROWS_PER_LINE) * D
            acc = jnp.zeros((L,), jnp.float32)
            for d in range(D):
                sv = plsc.load_gather(srows_v, [rloc, scol0 + d])
                pv = plsc.load_gather(prows_v, [rloc, pcol0 + d])
                acc = acc + sv * pv
            out_v[pl.ds(w * WAVE * IDXW + g * L, L)] = acc
            return carry

        lax.fori_loop(0, WAVE * IDXW // L, group_body, 0)

    for c in bias_copies:
        c.wait()

    def add_bias(g, carry):
        sl = pl.ds(g * L, L)
        out_v[sl] = out_v[sl] + sb_v[sl] + pb_v[sl] + jnp.float32(GLOBAL_MEAN)
        return carry

    lax.fori_loop(0, BPW // L, add_bias, 0)

    pltpu.sync_copy(out_v, out_hbm.at[pl.ds(base, BPW)])


def kernel(scientist_ids, paper_ids, scientist_factors, paper_factors,
           scientist_bias, paper_bias):
    sid3 = scientist_ids.reshape(16, 8, 128)
    pid3 = paper_ids.reshape(16, 8, 128)
    sf128 = scientist_factors.reshape(NLINES, 128)
    pf128 = paper_factors.reshape(NLINES, 128)
    sb = scientist_bias.reshape(-1)
    pb = paper_bias.reshape(-1)
    return _svdpp_sc(sid3, pid3, sf128, pf128, sb, pb)


# speedup vs baseline: 2.1591x; 2.1591x over previous
import functools

import jax
import jax.numpy as jnp
from jax import lax
from jax.experimental import pallas as pl
from jax.experimental.pallas import tpu as pltpu
from jax.experimental.pallas import tpu_sc as plsc

B = 16384
D = 32
TROWS = 8
NTILES = 125000
NC = 2
NS = 16
L = 16
NW = NC * NS
BPW = B // NW
IDXW = 128
NIDX = BPW // IDXW
WAVE = 32
NWAVES = BPW // WAVE
GLOBAL_MEAN = 3.82

_mesh = plsc.VectorSubcoreMesh(core_axis_name="c", subcore_axis_name="s")


@functools.partial(
    pl.kernel,
    mesh=_mesh,
    compiler_params=pltpu.CompilerParams(
        needs_layout_passes=False, use_tc_tiling_on_sc=True),
    out_type=jax.ShapeDtypeStruct((B,), jnp.float32),
    scratch_types=[
        pltpu.VMEM((NIDX, IDXW), jnp.int32),
        pltpu.VMEM((NIDX, IDXW), jnp.int32),
        pltpu.VMEM((L, TROWS, D), jnp.float32),
        pltpu.VMEM((L, TROWS, D), jnp.float32),
        pltpu.VMEM((BPW,), jnp.float32),
        pltpu.SemaphoreType.DMA,
    ],
)
def _factors_sc(sid_hbm, pid_hbm, sf_hbm, pf_hbm, out_hbm,
                sid_v, pid_v, st_v, pt_v, out_v, sem):
    wid = lax.axis_index("s") * NC + lax.axis_index("c")
    base = wid * BPW

    for j in range(NIDX):
        blk = wid * NIDX + j
        pltpu.sync_copy(sid_hbm.at[blk // 8, blk % 8], sid_v.at[j])
        pltpu.sync_copy(pid_hbm.at[blk // 8, blk % 8], pid_v.at[j])

    lanes = lax.iota(jnp.int32, L)
    NG = BPW // L

    def group_body(g, carry):
        j = g // (IDXW // L)
        o = (g - j * (IDXW // L)) * L
        ids_s = sid_v[j, pl.ds(o, L)]
        ids_p = pid_v[j, pl.ds(o, L)]
        sq = ids_s // TROWS
        pq = ids_p // TROWS
        for e in range(L):
            pltpu.async_copy(sf_hbm.at[sq[e]], st_v.at[e], sem)
            pltpu.async_copy(pf_hbm.at[pq[e]], pt_v.at[e], sem)
        for e in range(L):
            pltpu.make_async_copy(sf_hbm.at[0], st_v.at[0], sem).wait()
            pltpu.make_async_copy(pf_hbm.at[0], pt_v.at[0], sem).wait()
        srow = ids_s - sq * TROWS
        prow = ids_p - pq * TROWS
        acc = jnp.zeros((L,), jnp.float32)
        for d in range(D):
            dv = jnp.full((L,), d, jnp.int32)
            sv = plsc.load_gather(st_v, [lanes, srow, dv])
            pv = plsc.load_gather(pt_v, [lanes, prow, dv])
            acc = acc + sv * pv
        out_v[pl.ds(g * L, L)] = acc
        return carry

    lax.fori_loop(0, NG, group_body, 0)

    pltpu.sync_copy(out_v, out_hbm.at[pl.ds(base, BPW)])


@functools.partial(
    pl.kernel,
    mesh=_mesh,
    compiler_params=pltpu.CompilerParams(
        needs_layout_passes=False, use_tc_tiling_on_sc=False),
    out_type=jax.ShapeDtypeStruct((B,), jnp.float32),
    scratch_types=[
        pltpu.VMEM((NIDX, IDXW), jnp.int32),
        pltpu.VMEM((NIDX, IDXW), jnp.int32),
        pltpu.VMEM((BPW,), jnp.float32),
        pltpu.VMEM((BPW,), jnp.float32),
        pltpu.VMEM((BPW,), jnp.float32),
        pltpu.SemaphoreType.DMA,
    ],
)
def _bias_sc(part_hbm, sid_hbm, pid_hbm, sb_hbm, pb_hbm, out_hbm,
             sid_v, pid_v, sb_v, pb_v, acc_v, sem):
    wid = lax.axis_index("s") * NC + lax.axis_index("c")
    base = wid * BPW

    pltpu.sync_copy(sid_hbm.at[wid], sid_v)
    pltpu.sync_copy(pid_hbm.at[wid], pid_v)
    pltpu.sync_copy(part_hbm.at[pl.ds(base, BPW)], acc_v)

    copies = []
    for j in range(NIDX):
        rows = pl.ds(j * IDXW, IDXW)
        copies.append(pltpu.async_copy(sb_hbm.at[0].at[sid_v.at[j]],
                                       sb_v.at[rows], sem))
        copies.append(pltpu.async_copy(pb_hbm.at[0].at[pid_v.at[j]],
                                       pb_v.at[rows], sem))
    for c in copies:
        c.wait()

    def body(g, carry):
        sl = pl.ds(g * L, L)
        acc_v[sl] = acc_v[sl] + sb_v[sl] + pb_v[sl] + jnp.float32(GLOBAL_MEAN)
        return carry

    lax.fori_loop(0, BPW // L, body, 0)

    pltpu.sync_copy(acc_v, out_hbm.at[pl.ds(base, BPW)])


def kernel(scientist_ids, paper_ids, scientist_factors, paper_factors,
           scientist_bias, paper_bias):
    sid3 = scientist_ids.reshape(16, 8, 128)
    pid3 = paper_ids.reshape(16, 8, 128)
    sf3 = scientist_factors.reshape(NTILES, TROWS, D)
    pf3 = paper_factors.reshape(NTILES, TROWS, D)
    part = _factors_sc(sid3, pid3, sf3, pf3)
    sid4 = scientist_ids.reshape(NW, NIDX, IDXW)
    pid4 = paper_ids.reshape(NW, NIDX, IDXW)
    sb = scientist_bias.T
    pb = paper_bias.T
    return _bias_sc(part, sid4, pid4, sb, pb)
